# baseline (speedup 1.0000x reference)
import jax
import jax.numpy as jnp
from jax.experimental import pallas as pl
from jax.experimental.pallas import tpu as pltpu

B, T = 4, 4
S, H = 4096, 768
P = 128
NUM_TRAITS = 12
HH = H // 2
CHUNK = 2048
N = B * S // CHUNK
NBUF = 3


def _fused_kernel(idx_ref, table_ref, wp_ref, bp_ref, w1_ref, b1_ref,
                  w2_ref, b2_ref, hs_hbm, out_hbm,
                  in_buf, out_buf, gates_ref, in_sems, out_sems):
    def in_copy(i):
        return pltpu.make_async_copy(
            hs_hbm.at[pl.ds(i * CHUNK, CHUNK), :],
            in_buf.at[i % NBUF],
            in_sems.at[i % NBUF])

    def out_copy(i):
        return pltpu.make_async_copy(
            out_buf.at[i % NBUF],
            out_hbm.at[pl.ds(i * CHUNK, CHUNK), :],
            out_sems.at[i % NBUF])

    for k in range(min(NBUF, N)):
        in_copy(k).start()

    iota_k = jax.lax.broadcasted_iota(jnp.int32, (B, NUM_TRAITS), 1)
    acc = jnp.zeros((B, NUM_TRAITS), jnp.float32)
    for t in range(T):
        acc = acc + (idx_ref[:, t][:, None] == iota_k).astype(jnp.float32)
    pooled = acc * (1.0 / T)
    pv = jnp.dot(pooled, table_ref[...],
                 preferred_element_type=jnp.float32)
    h = jnp.dot(pv, wp_ref[...],
                preferred_element_type=jnp.float32) + bp_ref[...]
    g = jnp.tanh(jnp.dot(h, w1_ref[...],
                         preferred_element_type=jnp.float32) + b1_ref[...])
    gates_ref[...] = jax.nn.sigmoid(
        jnp.dot(g, w2_ref[...],
                preferred_element_type=jnp.float32) + b2_ref[...])

    for i in range(N):
        slot = i % NBUF
        in_copy(i).wait()
        if i >= NBUF:
            out_copy(i - NBUF).wait()
        b = (i * CHUNK) // S
        out_buf[slot] = in_buf[slot] * gates_ref[b:b + 1, :]
        out_copy(i).start()
        if i + NBUF < N:
            in_copy(i + NBUF).start()

    for j in range(max(N - NBUF, 0), N):
        out_copy(j).wait()


def kernel(trait_indices, hidden_states, trait_table, W_proj, b_proj,
           W1, b1, W2, b2):
    whole = lambda *_: (0, 0)
    hs2d = hidden_states.reshape(B * S, H)
    out2d = pl.pallas_call(
        _fused_kernel,
        in_specs=[
            pl.BlockSpec((B, T), whole),
            pl.BlockSpec((NUM_TRAITS, P), whole),
            pl.BlockSpec((P, H), whole),
            pl.BlockSpec((1, H), whole),
            pl.BlockSpec((H, HH), whole),
            pl.BlockSpec((1, HH), whole),
            pl.BlockSpec((HH, H), whole),
            pl.BlockSpec((1, H), whole),
            pl.BlockSpec(memory_space=pltpu.MemorySpace.HBM),
        ],
        out_specs=pl.BlockSpec(memory_space=pltpu.MemorySpace.HBM),
        out_shape=jax.ShapeDtypeStruct((B * S, H), jnp.float32),
        scratch_shapes=[
            pltpu.VMEM((NBUF, CHUNK, H), jnp.float32),
            pltpu.VMEM((NBUF, CHUNK, H), jnp.float32),
            pltpu.VMEM((B, H), jnp.float32),
            pltpu.SemaphoreType.DMA((NBUF,)),
            pltpu.SemaphoreType.DMA((NBUF,)),
        ],
    )(
        trait_indices.astype(jnp.int32),
        trait_table,
        W_proj,
        b_proj.reshape(1, H),
        W1,
        b1.reshape(1, HH),
        W2,
        b2.reshape(1, H),
        hs2d,
    )
    return out2d.reshape(B, S, H)


# speedup vs baseline: 1.0151x; 1.0151x over previous
import jax
import jax.numpy as jnp
from jax.experimental import pallas as pl
from jax.experimental.pallas import tpu as pltpu

B, T = 4, 4
S, H = 4096, 768
P = 128
NUM_TRAITS = 12
HH = H // 2
CHUNK = 4096
N = B * S // CHUNK
NBUF = 2


def _fused_kernel(idx_ref, table_ref, wp_ref, bp_ref, w1_ref, b1_ref,
                  w2_ref, b2_ref, hs_hbm, out_hbm,
                  in_buf, out_buf, gates_ref, in_sems, out_sems):
    def in_copy(i):
        return pltpu.make_async_copy(
            hs_hbm.at[pl.ds(i * CHUNK, CHUNK), :],
            in_buf.at[i % NBUF],
            in_sems.at[i % NBUF])

    def out_copy(i):
        return pltpu.make_async_copy(
            out_buf.at[i % NBUF],
            out_hbm.at[pl.ds(i * CHUNK, CHUNK), :],
            out_sems.at[i % NBUF])

    for k in range(min(NBUF, N)):
        in_copy(k).start()

    iota_k = jax.lax.broadcasted_iota(jnp.int32, (B, NUM_TRAITS), 1)
    acc = jnp.zeros((B, NUM_TRAITS), jnp.float32)
    for t in range(T):
        acc = acc + (idx_ref[:, t][:, None] == iota_k).astype(jnp.float32)
    pooled = acc * (1.0 / T)
    pv = jnp.dot(pooled, table_ref[...],
                 preferred_element_type=jnp.float32)
    h = jnp.dot(pv, wp_ref[...],
                preferred_element_type=jnp.float32) + bp_ref[...]
    g = jnp.tanh(jnp.dot(h, w1_ref[...],
                         preferred_element_type=jnp.float32) + b1_ref[...])
    gates_ref[...] = jax.nn.sigmoid(
        jnp.dot(g, w2_ref[...],
                preferred_element_type=jnp.float32) + b2_ref[...])

    for i in range(N):
        slot = i % NBUF
        in_copy(i).wait()
        if i >= NBUF:
            out_copy(i - NBUF).wait()
        b = (i * CHUNK) // S
        out_buf[slot] = in_buf[slot] * gates_ref[b:b + 1, :]
        out_copy(i).start()
        if i + NBUF < N:
            in_copy(i + NBUF).start()

    for j in range(max(N - NBUF, 0), N):
        out_copy(j).wait()


def kernel(trait_indices, hidden_states, trait_table, W_proj, b_proj,
           W1, b1, W2, b2):
    whole = lambda *_: (0, 0)
    hs2d = hidden_states.reshape(B * S, H)
    out2d = pl.pallas_call(
        _fused_kernel,
        in_specs=[
            pl.BlockSpec((B, T), whole),
            pl.BlockSpec((NUM_TRAITS, P), whole),
            pl.BlockSpec((P, H), whole),
            pl.BlockSpec((1, H), whole),
            pl.BlockSpec((H, HH), whole),
            pl.BlockSpec((1, HH), whole),
            pl.BlockSpec((HH, H), whole),
            pl.BlockSpec((1, H), whole),
            pl.BlockSpec(memory_space=pltpu.MemorySpace.HBM),
        ],
        out_specs=pl.BlockSpec(memory_space=pltpu.MemorySpace.HBM),
        out_shape=jax.ShapeDtypeStruct((B * S, H), jnp.float32),
        scratch_shapes=[
            pltpu.VMEM((NBUF, CHUNK, H), jnp.float32),
            pltpu.VMEM((NBUF, CHUNK, H), jnp.float32),
            pltpu.VMEM((B, H), jnp.float32),
            pltpu.SemaphoreType.DMA((NBUF,)),
            pltpu.SemaphoreType.DMA((NBUF,)),
        ],
    )(
        trait_indices.astype(jnp.int32),
        trait_table,
        W_proj,
        b_proj.reshape(1, H),
        W1,
        b1.reshape(1, HH),
        W2,
        b2.reshape(1, H),
        hs2d,
    )
    return out2d.reshape(B, S, H)
